# baseline (speedup 1.0000x reference)
import jax
import jax.numpy as jnp
from jax import lax
from jax.experimental import pallas as pl
from jax.experimental.pallas import tpu as pltpu
from jax.experimental.pallas import tpu_sc as plsc

TOP, LEFT, BOTTOM, RIGHT = 0, 1, 2, 3
REDUCTION = 16.0
B, T, R, C = 8, 100, 5000, 81
RP = 5120
SUB = 40
NW = 32
CHUNK = 1280
NBLK = CHUNK // 16
CP = 96
NPOS_CAP = 128
NNEG_CAP = 384
LN2 = 0.6931471805599453
SQRT2 = 1.4142135623730951


def _match_kernel(nms_ref, bb_ref, cl_ref, mpos_out, mneg_out, bcls_out, bbox_out,
                  cnt_out):
    a_t = nms_ref[0, TOP]
    a_l = nms_ref[0, LEFT]
    a_b = nms_ref[0, BOTTOM]
    a_r = nms_ref[0, RIGHT]
    area_a = jnp.maximum(a_b - a_t, 0.0) * jnp.maximum(a_r - a_l, 0.0)

    def iou_step(t, carry):
        best_iou, best_cls, bb_t, bb_l, bb_b, bb_r = carry
        g_t = bb_ref[0, t, TOP]
        g_l = bb_ref[0, t, LEFT]
        g_b = bb_ref[0, t, BOTTOM]
        g_r = bb_ref[0, t, RIGHT]
        area_b = jnp.maximum(g_b - g_t, 0.0) * jnp.maximum(g_r - g_l, 0.0)
        it = jnp.maximum(a_t, g_t)
        il = jnp.maximum(a_l, g_l)
        ib = jnp.minimum(a_b, g_b)
        ir = jnp.minimum(a_r, g_r)
        inter = jnp.maximum(ib - it, 0.0) * jnp.maximum(ir - il, 0.0)
        union = area_a + area_b - inter
        iou = inter / jnp.maximum(union, 1e-8)
        upd = iou > best_iou
        return (jnp.where(upd, iou, best_iou),
                jnp.where(upd, cl_ref[0, 0, t], best_cls),
                jnp.where(upd, g_t, bb_t),
                jnp.where(upd, g_l, bb_l),
                jnp.where(upd, g_b, bb_b),
                jnp.where(upd, g_r, bb_r))

    init = (jnp.full((SUB, 128), -1.0, jnp.float32),
            jnp.zeros((SUB, 128), jnp.int32),
            jnp.zeros((SUB, 128), jnp.float32),
            jnp.zeros((SUB, 128), jnp.float32),
            jnp.zeros((SUB, 128), jnp.float32),
            jnp.zeros((SUB, 128), jnp.float32))
    best_iou, best_cls, bb_t, bb_l, bb_b, bb_r = lax.fori_loop(0, T, iou_step, init)

    row = lax.broadcasted_iota(jnp.int32, (SUB, 128), 0)
    col = lax.broadcasted_iota(jnp.int32, (SUB, 128), 1)
    valid = (row * 128 + col) < R
    is_pos = best_iou > 0.5
    mp = (is_pos & valid).astype(jnp.int32)
    mn = ((~is_pos) & valid).astype(jnp.int32)
    mpos_out[0] = mp
    mneg_out[0] = mn
    bcls_out[0] = best_cls
    bbox_out[0, TOP] = bb_t
    bbox_out[0, LEFT] = bb_l
    bbox_out[0, BOTTOM] = bb_b
    bbox_out[0, RIGHT] = bb_r
    q = SUB // 4
    for i in range(4):
        cnt_out[0, 0, 2 * i] = jnp.sum(mp[i * q:(i + 1) * q, :].astype(jnp.float32))
        cnt_out[0, 0, 2 * i + 1] = jnp.sum(mn[i * q:(i + 1) * q, :].astype(jnp.float32))


def _match_stage(nms_t, bboxes, classes):
    return pl.pallas_call(
        _match_kernel,
        grid=(B,),
        in_specs=[
            pl.BlockSpec((1, 4, SUB, 128), lambda b: (b, 0, 0, 0)),
            pl.BlockSpec((1, T, 4), lambda b: (b, 0, 0), memory_space=pltpu.SMEM),
            pl.BlockSpec((1, 1, T), lambda b: (b, 0, 0), memory_space=pltpu.SMEM),
        ],
        out_specs=[
            pl.BlockSpec((1, SUB, 128), lambda b: (b, 0, 0)),
            pl.BlockSpec((1, SUB, 128), lambda b: (b, 0, 0)),
            pl.BlockSpec((1, SUB, 128), lambda b: (b, 0, 0)),
            pl.BlockSpec((1, 4, SUB, 128), lambda b: (b, 0, 0, 0)),
            pl.BlockSpec((1, 1, 8), lambda b: (b, 0, 0), memory_space=pltpu.SMEM),
        ],
        out_shape=[
            jax.ShapeDtypeStruct((B, SUB, 128), jnp.int32),
            jax.ShapeDtypeStruct((B, SUB, 128), jnp.int32),
            jax.ShapeDtypeStruct((B, SUB, 128), jnp.int32),
            jax.ShapeDtypeStruct((B, 4, SUB, 128), jnp.float32),
            jax.ShapeDtypeStruct((B, 1, 8), jnp.float32),
        ],
    )(nms_t, bboxes, classes.reshape(B, 1, T))


def _vlog(x):
    bits = plsc.bitcast(x, jnp.int32)
    e = ((bits >> 23) & 0xFF) - 127
    m = plsc.bitcast((bits & 0x7FFFFF) | 0x3F800000, jnp.float32)
    big = m > SQRT2
    m = jnp.where(big, m * 0.5, m)
    e = jnp.where(big, e + 1, e)
    s = (m - 1.0) / (m + 1.0)
    s2 = s * s
    p = 2.0 * s * (1.0 + s2 * (1.0 / 3.0 + s2 * (0.2 + s2 * (1.0 / 7.0 + s2 / 9.0))))
    return e.astype(jnp.float32) * LN2 + p


def _floor16(v):
    t = v.astype(jnp.int32).astype(jnp.float32)
    return t - (v < t).astype(jnp.float32)


def _ceil16(v):
    t = v.astype(jnp.int32).astype(jnp.float32)
    return t + (v > t).astype(jnp.float32)


def _splat_i(val):
    return jnp.full((16,), val, jnp.int32)


def _sc_kernel(mpos_hbm, mneg_hbm, bcls_hbm, bbox_hbm, nms_hbm, reg_hbm, cls_hbm,
               cnts_hbm,
               out_hbm,
               mpos_v, mneg_v, bcls_v, bbox_v, nms_v, reg_v,
               pos_list, neg_list, allcnt_v, rows_v, out_v, sem):
    wid = lax.axis_index("s") * 2 + lax.axis_index("c")
    iota = lax.iota(jnp.int32, 16)

    pltpu.sync_copy(mpos_hbm.at[wid], mpos_v)
    pltpu.sync_copy(mneg_hbm.at[wid], mneg_v)
    pltpu.sync_copy(bcls_hbm.at[wid], bcls_v)
    pltpu.sync_copy(bbox_hbm.at[wid], bbox_v)
    pltpu.sync_copy(nms_hbm.at[wid], nms_v)
    pltpu.sync_copy(reg_hbm.at[wid], reg_v)
    pltpu.sync_copy(cnts_hbm, allcnt_v)

    def compact(i, carry):
        cp, cn = carry
        mp = mpos_v[pl.ds(i * 16, 16)]
        mn = mneg_v[pl.ds(i * 16, 16)]
        jvec = iota + i * 16
        cump = plsc.cumsum(mp)
        cumn = plsc.cumsum(mn)
        plsc.store_scatter(pos_list, [cp + cump - mp], jvec, mask=mp != 0)
        plsc.store_scatter(neg_list, [cn + cumn - mn], jvec, mask=mn != 0)
        return (cp + jnp.sum(mp), cn + jnp.sum(mn))

    cnt_p, cnt_n = lax.fori_loop(0, NBLK, compact, (jnp.int32(0), jnp.int32(0)))

    pos_off = jnp.int32(0)
    neg_off = jnp.int32(0)
    for w2 in range(NW):
        cnt_row = allcnt_v[w2]
        cp = cnt_row[0].astype(jnp.int32)
        cn = cnt_row[1].astype(jnp.int32)
        before = jnp.int32(w2) < wid
        pos_off = pos_off + jnp.where(before, cp, 0)
        neg_off = neg_off + jnp.where(before, cn, 0)

    take_p = jnp.clip(NPOS_CAP - pos_off, 0, cnt_p)
    take_n = jnp.clip(NNEG_CAP - neg_off, 0, cnt_n)

    base = (wid >> 2) * R + (wid & 3) * CHUNK

    def row_block(lst_ref, take, is_pos_blk):
        def body(i, carry):
            cls_s, acc_s, reg_s = carry
            g = i * 16 + iota
            active = g < take
            j16 = lst_ref[pl.ds(i * 16, 16)]
            sj = jnp.where(active, j16, 0)
            gidx = sj + base
            pltpu.async_copy(cls_hbm.at[gidx], rows_v, sem).wait()
            cls16 = plsc.load_gather(bcls_v, [sj]) if is_pos_blk else _splat_i(0)

            def col(c, cc):
                mx, se, am, xc = cc
                cv = plsc.load_gather(rows_v, [iota, _splat_i(0) + c])
                gt = cv > mx
                mx2 = jnp.where(gt, cv, mx)
                se2 = se * jnp.exp(mx - mx2) + jnp.exp(cv - mx2)
                am2 = jnp.where(gt, c, am)
                xc2 = jnp.where(cls16 == c, cv, xc)
                return (mx2, se2, am2, xc2)

            mx0 = jnp.full((16,), -3.0e38, jnp.float32)
            z = jnp.zeros((16,), jnp.float32)
            mx, se, am, xc = lax.fori_loop(0, C, col, (mx0, z, _splat_i(0), z))
            lp = xc - mx - _vlog(se)
            af = jnp.where(active, 1.0, 0.0)
            cls_s = cls_s + jnp.sum(af * lp)
            hit = (am == cls16).astype(jnp.float32)
            acc_s = acc_s + jnp.sum(af * hit)

            if is_pos_blk:
                for c4, is_ceil in ((TOP, 0), (LEFT, 0), (BOTTOM, 1), (RIGHT, 1)):
                    c4v = _splat_i(c4)
                    nv = plsc.load_gather(nms_v, [sj, c4v]) * REDUCTION
                    rounded = (_ceil16(nv) if is_ceil else _floor16(nv)) / REDUCTION
                    rg = plsc.load_gather(reg_v, [sj, c4v])
                    bb = plsc.load_gather(bbox_v, [sj, c4v])
                    d = jnp.abs(rg - (bb - rounded))
                    term = jnp.where(d < 1.0, 0.5 * d * d, d - 0.5)
                    reg_s = reg_s + jnp.sum(af * term)
            return (cls_s, acc_s, reg_s)

        nblocks = (take + 15) >> 4
        z3 = (jnp.float32(0.0), jnp.float32(0.0), jnp.float32(0.0))
        return lax.fori_loop(0, nblocks, body, z3)

    pc, pa, pr = row_block(pos_list, take_p, True)
    nc, na, _ = row_block(neg_list, take_n, False)

    sums = (jnp.where(iota == 0, pc + nc,
            jnp.where(iota == 1, pa + na,
            jnp.where(iota == 2, pr, 0.0))))
    out_v[...] = sums
    pltpu.sync_copy(out_v, out_hbm.at[wid])


def _sc_stage(mpos, mneg, bcls, bbox, nms_p, reg_p, cls_p, cnts16):
    mesh = plsc.VectorSubcoreMesh(core_axis_name="c", subcore_axis_name="s")
    import functools
    k = functools.partial(
        pl.kernel,
        out_type=jax.ShapeDtypeStruct((NW, 16), jnp.float32),
        mesh=mesh,
        compiler_params=pltpu.CompilerParams(
            needs_layout_passes=False, use_tc_tiling_on_sc=False),
        scratch_types=[
            pltpu.VMEM((CHUNK,), jnp.int32),
            pltpu.VMEM((CHUNK,), jnp.int32),
            pltpu.VMEM((CHUNK,), jnp.int32),
            pltpu.VMEM((CHUNK, 4), jnp.float32),
            pltpu.VMEM((CHUNK, 4), jnp.float32),
            pltpu.VMEM((CHUNK, 4), jnp.float32),
            pltpu.VMEM((CHUNK,), jnp.int32),
            pltpu.VMEM((CHUNK,), jnp.int32),
            pltpu.VMEM((NW, 16), jnp.float32),
            pltpu.VMEM((16, CP), jnp.float32),
            pltpu.VMEM((16,), jnp.float32),
            pltpu.SemaphoreType.DMA,
        ],
    )(_sc_kernel)
    return k(mpos, mneg, bcls, bbox, nms_p, reg_p, cls_p, cnts16)


def _final_kernel(part_ref, cnt_ref, cls_out, reg_out, acc_out):
    part = part_ref[...]
    cnt = cnt_ref[...]
    col = lax.broadcasted_iota(jnp.int32, (NW, 16), 1)
    tcls = jnp.sum(jnp.where(col == 0, part, 0.0))
    tacc = jnp.sum(jnp.where(col == 1, part, 0.0))
    treg = jnp.sum(jnp.where(col == 2, part, 0.0))
    tot_p = jnp.sum(jnp.where(col == 0, cnt, 0.0))
    tot_n = jnp.sum(jnp.where(col == 1, cnt, 0.0))
    n_pos = jnp.minimum(tot_p, float(NPOS_CAP))
    n_sel = n_pos + jnp.minimum(tot_n, float(NNEG_CAP))
    cls_out[0, 0] = -tcls / n_sel
    acc_out[0, 0] = tacc / n_sel
    rl = treg / jnp.maximum(n_pos, 1.0) / 4.0
    reg_out[0, 0] = jnp.where(n_pos > 0.0, rl, 0.0)


def _final_stage(partials, cnts16):
    return pl.pallas_call(
        _final_kernel,
        in_specs=[
            pl.BlockSpec((NW, 16), lambda: (0, 0)),
            pl.BlockSpec((NW, 16), lambda: (0, 0)),
        ],
        out_specs=[
            pl.BlockSpec((1, 1), lambda: (0, 0), memory_space=pltpu.SMEM),
            pl.BlockSpec((1, 1), lambda: (0, 0), memory_space=pltpu.SMEM),
            pl.BlockSpec((1, 1), lambda: (0, 0), memory_space=pltpu.SMEM),
        ],
        out_shape=[jax.ShapeDtypeStruct((1, 1), jnp.float32)] * 3,
    )(partials, cnts16)


@jax.jit
def kernel(nms_reg, nms_cls, rcnn_reg, rcnn_cls, bboxes, classes):
    del nms_cls
    pad = ((0, 0), (0, 0), (0, RP - R))
    nms_t = jnp.pad(jnp.transpose(nms_reg, (0, 2, 1)), pad).reshape(B, 4, SUB, 128)

    mpos, mneg, bcls, bbox, cnts = _match_stage(nms_t, bboxes, classes)

    mpos = mpos.reshape(NW, CHUNK)
    mneg = mneg.reshape(NW, CHUNK)
    bcls = bcls.reshape(NW, CHUNK)
    bbox = jnp.transpose(bbox.reshape(B, 4, RP), (0, 2, 1)).reshape(NW, CHUNK, 4)
    rpad = ((0, 0), (0, RP - R), (0, 0))
    nms_p = jnp.pad(nms_reg, rpad).reshape(NW, CHUNK, 4)
    reg_p = jnp.pad(rcnn_reg, rpad).reshape(NW, CHUNK, 4)
    cls_p = jnp.pad(rcnn_cls.reshape(B * R, C), ((0, 0), (0, CP - C)))
    cnts16 = jnp.pad(cnts.reshape(NW, 2), ((0, 0), (0, 14)))

    partials = _sc_stage(mpos, mneg, bcls, bbox, nms_p, reg_p, cls_p, cnts16)
    cls_l, reg_l, acc_l = _final_stage(partials, cnts16)
    return (cls_l.reshape(1), reg_l.reshape(1), acc_l.reshape(1))


# speedup vs baseline: 1.1159x; 1.1159x over previous
import jax
import jax.numpy as jnp
from jax import lax
from jax.experimental import pallas as pl
from jax.experimental.pallas import tpu as pltpu
from jax.experimental.pallas import tpu_sc as plsc

TOP, LEFT, BOTTOM, RIGHT = 0, 1, 2, 3
REDUCTION = 16.0
B, T, R, C = 8, 100, 5000, 81
RP = 5120
SUB = 40
NW = 32
CHUNK = 1280
NBLK = CHUNK // 16
FLAT_ROWS = B * R * C // 16
NPOS_CAP = 128
NNEG_CAP = 384
LN2 = 0.6931471805599453
SQRT2 = 1.4142135623730951


def _match_kernel(nms_ref, bb_ref, cl_ref, mpos_out, mneg_out, bcls_out, bbox_out,
                  cnt_out):
    a_t = nms_ref[0, TOP]
    a_l = nms_ref[0, LEFT]
    a_b = nms_ref[0, BOTTOM]
    a_r = nms_ref[0, RIGHT]
    area_a = jnp.maximum(a_b - a_t, 0.0) * jnp.maximum(a_r - a_l, 0.0)

    def iou_step(t, carry):
        best_iou, best_cls, bb_t, bb_l, bb_b, bb_r = carry
        g_t = bb_ref[0, t, TOP]
        g_l = bb_ref[0, t, LEFT]
        g_b = bb_ref[0, t, BOTTOM]
        g_r = bb_ref[0, t, RIGHT]
        area_b = jnp.maximum(g_b - g_t, 0.0) * jnp.maximum(g_r - g_l, 0.0)
        it = jnp.maximum(a_t, g_t)
        il = jnp.maximum(a_l, g_l)
        ib = jnp.minimum(a_b, g_b)
        ir = jnp.minimum(a_r, g_r)
        inter = jnp.maximum(ib - it, 0.0) * jnp.maximum(ir - il, 0.0)
        union = area_a + area_b - inter
        iou = inter / jnp.maximum(union, 1e-8)
        upd = iou > best_iou
        return (jnp.where(upd, iou, best_iou),
                jnp.where(upd, cl_ref[0, 0, t], best_cls),
                jnp.where(upd, g_t, bb_t),
                jnp.where(upd, g_l, bb_l),
                jnp.where(upd, g_b, bb_b),
                jnp.where(upd, g_r, bb_r))

    init = (jnp.full((SUB, 128), -1.0, jnp.float32),
            jnp.zeros((SUB, 128), jnp.int32),
            jnp.zeros((SUB, 128), jnp.float32),
            jnp.zeros((SUB, 128), jnp.float32),
            jnp.zeros((SUB, 128), jnp.float32),
            jnp.zeros((SUB, 128), jnp.float32))
    best_iou, best_cls, bb_t, bb_l, bb_b, bb_r = lax.fori_loop(0, T, iou_step, init)

    row = lax.broadcasted_iota(jnp.int32, (SUB, 128), 0)
    col = lax.broadcasted_iota(jnp.int32, (SUB, 128), 1)
    valid = (row * 128 + col) < R
    is_pos = best_iou > 0.5
    mp = (is_pos & valid).astype(jnp.int32)
    mn = ((~is_pos) & valid).astype(jnp.int32)
    mpos_out[0] = mp
    mneg_out[0] = mn
    bcls_out[0] = best_cls
    bbox_out[0, TOP] = bb_t
    bbox_out[0, LEFT] = bb_l
    bbox_out[0, BOTTOM] = bb_b
    bbox_out[0, RIGHT] = bb_r
    q = SUB // 4
    for i in range(4):
        cnt_out[0, 0, 2 * i] = jnp.sum(mp[i * q:(i + 1) * q, :].astype(jnp.float32))
        cnt_out[0, 0, 2 * i + 1] = jnp.sum(mn[i * q:(i + 1) * q, :].astype(jnp.float32))


def _match_stage(nms_t, bboxes, classes):
    return pl.pallas_call(
        _match_kernel,
        grid=(B,),
        in_specs=[
            pl.BlockSpec((1, 4, SUB, 128), lambda b: (b, 0, 0, 0)),
            pl.BlockSpec((1, T, 4), lambda b: (b, 0, 0), memory_space=pltpu.SMEM),
            pl.BlockSpec((1, 1, T), lambda b: (b, 0, 0), memory_space=pltpu.SMEM),
        ],
        out_specs=[
            pl.BlockSpec((1, SUB, 128), lambda b: (b, 0, 0)),
            pl.BlockSpec((1, SUB, 128), lambda b: (b, 0, 0)),
            pl.BlockSpec((1, SUB, 128), lambda b: (b, 0, 0)),
            pl.BlockSpec((1, 4, SUB, 128), lambda b: (b, 0, 0, 0)),
            pl.BlockSpec((1, 1, 8), lambda b: (b, 0, 0), memory_space=pltpu.SMEM),
        ],
        out_shape=[
            jax.ShapeDtypeStruct((B, SUB, 128), jnp.int32),
            jax.ShapeDtypeStruct((B, SUB, 128), jnp.int32),
            jax.ShapeDtypeStruct((B, SUB, 128), jnp.int32),
            jax.ShapeDtypeStruct((B, 4, SUB, 128), jnp.float32),
            jax.ShapeDtypeStruct((B, 1, 8), jnp.float32),
        ],
    )(nms_t, bboxes, classes.reshape(B, 1, T))


def _vlog(x):
    bits = plsc.bitcast(x, jnp.int32)
    e = ((bits >> 23) & 0xFF) - 127
    m = plsc.bitcast((bits & 0x7FFFFF) | 0x3F800000, jnp.float32)
    big = m > SQRT2
    m = jnp.where(big, m * 0.5, m)
    e = jnp.where(big, e + 1, e)
    s = (m - 1.0) / (m + 1.0)
    s2 = s * s
    p = 2.0 * s * (1.0 + s2 * (1.0 / 3.0 + s2 * (0.2 + s2 * (1.0 / 7.0 + s2 / 9.0))))
    return e.astype(jnp.float32) * LN2 + p


def _floor16(v):
    t = v.astype(jnp.int32).astype(jnp.float32)
    return t - (v < t).astype(jnp.float32)


def _ceil16(v):
    t = v.astype(jnp.int32).astype(jnp.float32)
    return t + (v > t).astype(jnp.float32)


def _splat_i(val):
    return jnp.full((16,), val, jnp.int32)


def _sc_kernel(mpos_hbm, mneg_hbm, bcls_hbm, bbox_hbm, nms_hbm, reg_hbm, cls_hbm,
               cnts_hbm,
               out_hbm,
               mpos_v, mneg_v, bcls_v, bbox_v, nms_v, reg_v,
               pos_list, neg_list, allcnt_v, rows_v, out_v, sem):
    wid = lax.axis_index("s") * 2 + lax.axis_index("c")
    iota = lax.iota(jnp.int32, 16)

    pltpu.sync_copy(mpos_hbm.at[wid], mpos_v)
    pltpu.sync_copy(mneg_hbm.at[wid], mneg_v)
    pltpu.sync_copy(bcls_hbm.at[wid], bcls_v)
    pltpu.sync_copy(bbox_hbm.at[wid], bbox_v)
    pltpu.sync_copy(nms_hbm.at[wid], nms_v)
    pltpu.sync_copy(reg_hbm.at[wid], reg_v)
    pltpu.sync_copy(cnts_hbm, allcnt_v)

    def compact(i, carry):
        cp, cn = carry
        mp = mpos_v[pl.ds(i * 16, 16)]
        mn = mneg_v[pl.ds(i * 16, 16)]
        jvec = iota + i * 16
        cump = plsc.cumsum(mp)
        cumn = plsc.cumsum(mn)
        plsc.store_scatter(pos_list, [cp + cump - mp], jvec, mask=mp != 0)
        plsc.store_scatter(neg_list, [cn + cumn - mn], jvec, mask=mn != 0)
        return (cp + jnp.sum(mp), cn + jnp.sum(mn))

    cnt_p, cnt_n = lax.fori_loop(0, NBLK, compact, (jnp.int32(0), jnp.int32(0)))

    pos_off = jnp.int32(0)
    neg_off = jnp.int32(0)
    for w2 in range(NW):
        cnt_row = allcnt_v[w2]
        cp = cnt_row[0].astype(jnp.int32)
        cn = cnt_row[1].astype(jnp.int32)
        before = jnp.int32(w2) < wid
        pos_off = pos_off + jnp.where(before, cp, 0)
        neg_off = neg_off + jnp.where(before, cn, 0)

    take_p = jnp.clip(NPOS_CAP - pos_off, 0, cnt_p)
    take_n = jnp.clip(NNEG_CAP - neg_off, 0, cnt_n)

    base = (wid >> 2) * R + (wid & 3) * CHUNK

    def row_block(lst_ref, take, is_pos_blk):
        def body(i, carry):
            cls_s, acc_s, reg_s = carry
            g = i * 16 + iota
            active = g < take
            j16 = lst_ref[pl.ds(i * 16, 16)]
            sj = jnp.where(active, j16, 0)
            gidx = sj + base
            e = gidx * C
            r0 = e >> 4
            o = e & 15
            copies = [pltpu.async_copy(cls_hbm.at[r0 + k], rows_v.at[k], sem)
                      for k in range(6)]
            for cp_ in copies:
                cp_.wait()
            cls16 = plsc.load_gather(bcls_v, [sj]) if is_pos_blk else _splat_i(0)

            def col(c, cc):
                mx, se, am, xc = cc
                t = o + c
                cv = plsc.load_gather(rows_v, [t >> 4, iota, t & 15])
                gt = cv > mx
                mx2 = jnp.where(gt, cv, mx)
                se2 = se * jnp.exp(mx - mx2) + jnp.exp(cv - mx2)
                am2 = jnp.where(gt, c, am)
                xc2 = jnp.where(cls16 == c, cv, xc)
                return (mx2, se2, am2, xc2)

            mx0 = jnp.full((16,), -3.0e38, jnp.float32)
            z = jnp.zeros((16,), jnp.float32)
            mx, se, am, xc = lax.fori_loop(0, C, col, (mx0, z, _splat_i(0), z))
            lp = xc - mx - _vlog(se)
            af = jnp.where(active, 1.0, 0.0)
            cls_s = cls_s + jnp.sum(af * lp)
            hit = (am == cls16).astype(jnp.float32)
            acc_s = acc_s + jnp.sum(af * hit)

            if is_pos_blk:
                for c4, is_ceil in ((TOP, 0), (LEFT, 0), (BOTTOM, 1), (RIGHT, 1)):
                    c4v = _splat_i(c4)
                    nv = plsc.load_gather(nms_v, [sj, c4v]) * REDUCTION
                    rounded = (_ceil16(nv) if is_ceil else _floor16(nv)) / REDUCTION
                    rg = plsc.load_gather(reg_v, [sj, c4v])
                    bb = plsc.load_gather(bbox_v, [sj, c4v])
                    d = jnp.abs(rg - (bb - rounded))
                    term = jnp.where(d < 1.0, 0.5 * d * d, d - 0.5)
                    reg_s = reg_s + jnp.sum(af * term)
            return (cls_s, acc_s, reg_s)

        nblocks = (take + 15) >> 4
        z3 = (jnp.float32(0.0), jnp.float32(0.0), jnp.float32(0.0))
        return lax.fori_loop(0, nblocks, body, z3)

    pc, pa, pr = row_block(pos_list, take_p, True)
    nc, na, _ = row_block(neg_list, take_n, False)

    sums = (jnp.where(iota == 0, pc + nc,
            jnp.where(iota == 1, pa + na,
            jnp.where(iota == 2, pr, 0.0))))
    out_v[...] = sums
    pltpu.sync_copy(out_v, out_hbm.at[wid])


def _sc_stage(mpos, mneg, bcls, bbox, nms_p, reg_p, cls_p, cnts16):
    mesh = plsc.VectorSubcoreMesh(core_axis_name="c", subcore_axis_name="s")
    import functools
    k = functools.partial(
        pl.kernel,
        out_type=jax.ShapeDtypeStruct((NW, 16), jnp.float32),
        mesh=mesh,
        compiler_params=pltpu.CompilerParams(
            needs_layout_passes=False, use_tc_tiling_on_sc=False),
        scratch_types=[
            pltpu.VMEM((CHUNK,), jnp.int32),
            pltpu.VMEM((CHUNK,), jnp.int32),
            pltpu.VMEM((CHUNK,), jnp.int32),
            pltpu.VMEM((CHUNK, 4), jnp.float32),
            pltpu.VMEM((CHUNK, 4), jnp.float32),
            pltpu.VMEM((CHUNK, 4), jnp.float32),
            pltpu.VMEM((CHUNK,), jnp.int32),
            pltpu.VMEM((CHUNK,), jnp.int32),
            pltpu.VMEM((NW, 16), jnp.float32),
            pltpu.VMEM((6, 16, 16), jnp.float32),
            pltpu.VMEM((16,), jnp.float32),
            pltpu.SemaphoreType.DMA,
        ],
    )(_sc_kernel)
    return k(mpos, mneg, bcls, bbox, nms_p, reg_p, cls_p, cnts16)


def _final_kernel(part_ref, cnt_ref, cls_out, reg_out, acc_out):
    part = part_ref[...]
    cnt = cnt_ref[...]
    col = lax.broadcasted_iota(jnp.int32, (NW, 16), 1)
    tcls = jnp.sum(jnp.where(col == 0, part, 0.0))
    tacc = jnp.sum(jnp.where(col == 1, part, 0.0))
    treg = jnp.sum(jnp.where(col == 2, part, 0.0))
    tot_p = jnp.sum(jnp.where(col == 0, cnt, 0.0))
    tot_n = jnp.sum(jnp.where(col == 1, cnt, 0.0))
    n_pos = jnp.minimum(tot_p, float(NPOS_CAP))
    n_sel = n_pos + jnp.minimum(tot_n, float(NNEG_CAP))
    cls_out[0, 0] = -tcls / n_sel
    acc_out[0, 0] = tacc / n_sel
    rl = treg / jnp.maximum(n_pos, 1.0) / 4.0
    reg_out[0, 0] = jnp.where(n_pos > 0.0, rl, 0.0)


def _final_stage(partials, cnts16):
    return pl.pallas_call(
        _final_kernel,
        in_specs=[
            pl.BlockSpec((NW, 16), lambda: (0, 0)),
            pl.BlockSpec((NW, 16), lambda: (0, 0)),
        ],
        out_specs=[
            pl.BlockSpec((1, 1), lambda: (0, 0), memory_space=pltpu.SMEM),
            pl.BlockSpec((1, 1), lambda: (0, 0), memory_space=pltpu.SMEM),
            pl.BlockSpec((1, 1), lambda: (0, 0), memory_space=pltpu.SMEM),
        ],
        out_shape=[jax.ShapeDtypeStruct((1, 1), jnp.float32)] * 3,
    )(partials, cnts16)


@jax.jit
def kernel(nms_reg, nms_cls, rcnn_reg, rcnn_cls, bboxes, classes):
    del nms_cls
    pad = ((0, 0), (0, 0), (0, RP - R))
    nms_t = jnp.pad(jnp.transpose(nms_reg, (0, 2, 1)), pad).reshape(B, 4, SUB, 128)

    mpos, mneg, bcls, bbox, cnts = _match_stage(nms_t, bboxes, classes)

    mpos = mpos.reshape(NW, CHUNK)
    mneg = mneg.reshape(NW, CHUNK)
    bcls = bcls.reshape(NW, CHUNK)
    bbox = jnp.transpose(bbox.reshape(B, 4, RP), (0, 2, 1)).reshape(NW, CHUNK, 4)
    rpad = ((0, 0), (0, RP - R), (0, 0))
    nms_p = jnp.pad(nms_reg, rpad).reshape(NW, CHUNK, 4)
    reg_p = jnp.pad(rcnn_reg, rpad).reshape(NW, CHUNK, 4)
    cls_p = rcnn_cls.reshape(FLAT_ROWS, 16)
    cnts16 = jnp.pad(cnts.reshape(NW, 2), ((0, 0), (0, 14)))

    partials = _sc_stage(mpos, mneg, bcls, bbox, nms_p, reg_p, cls_p, cnts16)
    cls_l, reg_l, acc_l = _final_stage(partials, cnts16)
    return (cls_l.reshape(1), reg_l.reshape(1), acc_l.reshape(1))


# speedup vs baseline: 1.1617x; 1.0410x over previous
import jax
import jax.numpy as jnp
from jax import lax
from jax.experimental import pallas as pl
from jax.experimental.pallas import tpu as pltpu
from jax.experimental.pallas import tpu_sc as plsc

TOP, LEFT, BOTTOM, RIGHT = 0, 1, 2, 3
REDUCTION = 16.0
B, T, R, C = 8, 100, 5000, 81
RP = 5120
SUB = 40
NW = 32
CHUNK = 1280
NBLK = CHUNK // 16
FLAT_ROWS = B * R * C // 16
NPOS_CAP = 128
NNEG_CAP = 384
LN2 = 0.6931471805599453
SQRT2 = 1.4142135623730951


def _match_kernel(nms_ref, bb_ref, cl_ref, mpos_out, mneg_out, bcls_out, bbox_out,
                  cnt_out):
    a_t = nms_ref[0, TOP]
    a_l = nms_ref[0, LEFT]
    a_b = nms_ref[0, BOTTOM]
    a_r = nms_ref[0, RIGHT]
    area_a = jnp.maximum(a_b - a_t, 0.0) * jnp.maximum(a_r - a_l, 0.0)

    def iou_step(t, carry):
        best_iou, best_cls, bb_t, bb_l, bb_b, bb_r = carry
        g_t = bb_ref[0, t, TOP]
        g_l = bb_ref[0, t, LEFT]
        g_b = bb_ref[0, t, BOTTOM]
        g_r = bb_ref[0, t, RIGHT]
        area_b = jnp.maximum(g_b - g_t, 0.0) * jnp.maximum(g_r - g_l, 0.0)
        it = jnp.maximum(a_t, g_t)
        il = jnp.maximum(a_l, g_l)
        ib = jnp.minimum(a_b, g_b)
        ir = jnp.minimum(a_r, g_r)
        inter = jnp.maximum(ib - it, 0.0) * jnp.maximum(ir - il, 0.0)
        union = area_a + area_b - inter
        iou = inter / jnp.maximum(union, 1e-8)
        upd = iou > best_iou
        return (jnp.where(upd, iou, best_iou),
                jnp.where(upd, cl_ref[0, 0, t], best_cls),
                jnp.where(upd, g_t, bb_t),
                jnp.where(upd, g_l, bb_l),
                jnp.where(upd, g_b, bb_b),
                jnp.where(upd, g_r, bb_r))

    carry = (jnp.full((SUB, 128), -1.0, jnp.float32),
             jnp.zeros((SUB, 128), jnp.int32),
             jnp.zeros((SUB, 128), jnp.float32),
             jnp.zeros((SUB, 128), jnp.float32),
             jnp.zeros((SUB, 128), jnp.float32),
             jnp.zeros((SUB, 128), jnp.float32))
    for t in range(T):
        carry = iou_step(t, carry)
    best_iou, best_cls, bb_t, bb_l, bb_b, bb_r = carry

    row = lax.broadcasted_iota(jnp.int32, (SUB, 128), 0)
    col = lax.broadcasted_iota(jnp.int32, (SUB, 128), 1)
    valid = (row * 128 + col) < R
    is_pos = best_iou > 0.5
    mp = (is_pos & valid).astype(jnp.int32)
    mn = ((~is_pos) & valid).astype(jnp.int32)
    mpos_out[0] = mp
    mneg_out[0] = mn
    bcls_out[0] = best_cls
    bbox_out[0, TOP] = bb_t
    bbox_out[0, LEFT] = bb_l
    bbox_out[0, BOTTOM] = bb_b
    bbox_out[0, RIGHT] = bb_r
    q = SUB // 4
    for i in range(4):
        cnt_out[0, 0, 2 * i] = jnp.sum(mp[i * q:(i + 1) * q, :].astype(jnp.float32))
        cnt_out[0, 0, 2 * i + 1] = jnp.sum(mn[i * q:(i + 1) * q, :].astype(jnp.float32))


def _match_stage(nms_t, bboxes, classes):
    return pl.pallas_call(
        _match_kernel,
        grid=(B,),
        in_specs=[
            pl.BlockSpec((1, 4, SUB, 128), lambda b: (b, 0, 0, 0)),
            pl.BlockSpec((1, T, 4), lambda b: (b, 0, 0), memory_space=pltpu.SMEM),
            pl.BlockSpec((1, 1, T), lambda b: (b, 0, 0), memory_space=pltpu.SMEM),
        ],
        out_specs=[
            pl.BlockSpec((1, SUB, 128), lambda b: (b, 0, 0)),
            pl.BlockSpec((1, SUB, 128), lambda b: (b, 0, 0)),
            pl.BlockSpec((1, SUB, 128), lambda b: (b, 0, 0)),
            pl.BlockSpec((1, 4, SUB, 128), lambda b: (b, 0, 0, 0)),
            pl.BlockSpec((1, 1, 8), lambda b: (b, 0, 0), memory_space=pltpu.SMEM),
        ],
        out_shape=[
            jax.ShapeDtypeStruct((B, SUB, 128), jnp.int32),
            jax.ShapeDtypeStruct((B, SUB, 128), jnp.int32),
            jax.ShapeDtypeStruct((B, SUB, 128), jnp.int32),
            jax.ShapeDtypeStruct((B, 4, SUB, 128), jnp.float32),
            jax.ShapeDtypeStruct((B, 1, 8), jnp.float32),
        ],
    )(nms_t, bboxes, classes.reshape(B, 1, T))


def _vlog(x):
    bits = plsc.bitcast(x, jnp.int32)
    e = ((bits >> 23) & 0xFF) - 127
    m = plsc.bitcast((bits & 0x7FFFFF) | 0x3F800000, jnp.float32)
    big = m > SQRT2
    m = jnp.where(big, m * 0.5, m)
    e = jnp.where(big, e + 1, e)
    s = (m - 1.0) / (m + 1.0)
    s2 = s * s
    p = 2.0 * s * (1.0 + s2 * (1.0 / 3.0 + s2 * (0.2 + s2 * (1.0 / 7.0 + s2 / 9.0))))
    return e.astype(jnp.float32) * LN2 + p


def _floor16(v):
    t = v.astype(jnp.int32).astype(jnp.float32)
    return t - (v < t).astype(jnp.float32)


def _ceil16(v):
    t = v.astype(jnp.int32).astype(jnp.float32)
    return t + (v > t).astype(jnp.float32)


def _splat_i(val):
    return jnp.full((16,), val, jnp.int32)


def _sc_kernel(mpos_hbm, mneg_hbm, bcls_hbm, bbox_hbm, nms_hbm, reg_hbm, cls_hbm,
               cnts_hbm,
               out_hbm,
               mpos_v, mneg_v, bcls_v, bbox_v, nms_v, reg_v,
               pos_list, neg_list, allcnt_v, rows_v, out_v, sem):
    wid = lax.axis_index("s") * 2 + lax.axis_index("c")
    iota = lax.iota(jnp.int32, 16)

    pltpu.sync_copy(mpos_hbm.at[wid], mpos_v)
    pltpu.sync_copy(mneg_hbm.at[wid], mneg_v)
    pltpu.sync_copy(bcls_hbm.at[wid], bcls_v)
    pltpu.sync_copy(bbox_hbm.at[wid], bbox_v)
    pltpu.sync_copy(nms_hbm.at[wid], nms_v)
    pltpu.sync_copy(reg_hbm.at[wid], reg_v)
    pltpu.sync_copy(cnts_hbm, allcnt_v)

    def compact(i, carry):
        cp, cn = carry
        mp = mpos_v[pl.ds(i * 16, 16)]
        mn = mneg_v[pl.ds(i * 16, 16)]
        jvec = iota + i * 16
        cump = plsc.cumsum(mp)
        cumn = plsc.cumsum(mn)
        plsc.store_scatter(pos_list, [cp + cump - mp], jvec, mask=mp != 0)
        plsc.store_scatter(neg_list, [cn + cumn - mn], jvec, mask=mn != 0)
        return (cp + jnp.sum(mp), cn + jnp.sum(mn))

    cnt_p, cnt_n = lax.fori_loop(0, NBLK, compact, (jnp.int32(0), jnp.int32(0)))

    pos_off = jnp.int32(0)
    neg_off = jnp.int32(0)
    for w2 in range(NW):
        cnt_row = allcnt_v[w2]
        cp = cnt_row[0].astype(jnp.int32)
        cn = cnt_row[1].astype(jnp.int32)
        before = jnp.int32(w2) < wid
        pos_off = pos_off + jnp.where(before, cp, 0)
        neg_off = neg_off + jnp.where(before, cn, 0)

    take_p = jnp.clip(NPOS_CAP - pos_off, 0, cnt_p)
    take_n = jnp.clip(NNEG_CAP - neg_off, 0, cnt_n)

    base = (wid >> 2) * R + (wid & 3) * CHUNK

    def row_block(lst_ref, take, is_pos_blk):
        def body(i, carry):
            cls_s, acc_s, reg_s = carry
            g = i * 16 + iota
            active = g < take
            j16 = lst_ref[pl.ds(i * 16, 16)]
            sj = jnp.where(active, j16, 0)
            gidx = sj + base
            e = gidx * C
            r0 = e >> 4
            o = e & 15
            copies = [pltpu.async_copy(cls_hbm.at[r0 + k], rows_v.at[k], sem)
                      for k in range(6)]
            for cp_ in copies:
                cp_.wait()
            cls16 = plsc.load_gather(bcls_v, [sj]) if is_pos_blk else _splat_i(0)

            def col(c, cc):
                mx, se, am, xc = cc
                t = o + c
                cv = plsc.load_gather(rows_v, [t >> 4, iota, t & 15])
                gt = cv > mx
                mx2 = jnp.where(gt, cv, mx)
                se2 = se * jnp.exp(mx - mx2) + jnp.exp(cv - mx2)
                am2 = jnp.where(gt, c, am)
                xc2 = jnp.where(cls16 == c, cv, xc)
                return (mx2, se2, am2, xc2)

            mx0 = jnp.full((16,), -3.0e38, jnp.float32)
            z = jnp.zeros((16,), jnp.float32)
            mx, se, am, xc = lax.fori_loop(0, C, col, (mx0, z, _splat_i(0), z))
            lp = xc - mx - _vlog(se)
            af = jnp.where(active, 1.0, 0.0)
            cls_s = cls_s + jnp.sum(af * lp)
            hit = (am == cls16).astype(jnp.float32)
            acc_s = acc_s + jnp.sum(af * hit)

            if is_pos_blk:
                for c4, is_ceil in ((TOP, 0), (LEFT, 0), (BOTTOM, 1), (RIGHT, 1)):
                    c4v = _splat_i(c4)
                    nv = plsc.load_gather(nms_v, [sj, c4v]) * REDUCTION
                    rounded = (_ceil16(nv) if is_ceil else _floor16(nv)) / REDUCTION
                    rg = plsc.load_gather(reg_v, [sj, c4v])
                    bb = plsc.load_gather(bbox_v, [sj, c4v])
                    d = jnp.abs(rg - (bb - rounded))
                    term = jnp.where(d < 1.0, 0.5 * d * d, d - 0.5)
                    reg_s = reg_s + jnp.sum(af * term)
            return (cls_s, acc_s, reg_s)

        nblocks = (take + 15) >> 4
        z3 = (jnp.float32(0.0), jnp.float32(0.0), jnp.float32(0.0))
        return lax.fori_loop(0, nblocks, body, z3)

    pc, pa, pr = row_block(pos_list, take_p, True)
    nc, na, _ = row_block(neg_list, take_n, False)

    sums = (jnp.where(iota == 0, pc + nc,
            jnp.where(iota == 1, pa + na,
            jnp.where(iota == 2, pr, 0.0))))
    out_v[...] = sums
    pltpu.sync_copy(out_v, out_hbm.at[wid])


def _sc_stage(mpos, mneg, bcls, bbox, nms_p, reg_p, cls_p, cnts16):
    mesh = plsc.VectorSubcoreMesh(core_axis_name="c", subcore_axis_name="s")
    import functools
    k = functools.partial(
        pl.kernel,
        out_type=jax.ShapeDtypeStruct((NW, 16), jnp.float32),
        mesh=mesh,
        compiler_params=pltpu.CompilerParams(
            needs_layout_passes=False, use_tc_tiling_on_sc=False),
        scratch_types=[
            pltpu.VMEM((CHUNK,), jnp.int32),
            pltpu.VMEM((CHUNK,), jnp.int32),
            pltpu.VMEM((CHUNK,), jnp.int32),
            pltpu.VMEM((CHUNK, 4), jnp.float32),
            pltpu.VMEM((CHUNK, 4), jnp.float32),
            pltpu.VMEM((CHUNK, 4), jnp.float32),
            pltpu.VMEM((CHUNK,), jnp.int32),
            pltpu.VMEM((CHUNK,), jnp.int32),
            pltpu.VMEM((NW, 16), jnp.float32),
            pltpu.VMEM((6, 16, 16), jnp.float32),
            pltpu.VMEM((16,), jnp.float32),
            pltpu.SemaphoreType.DMA,
        ],
    )(_sc_kernel)
    return k(mpos, mneg, bcls, bbox, nms_p, reg_p, cls_p, cnts16)


def _final_kernel(part_ref, cnt_ref, cls_out, reg_out, acc_out):
    part = part_ref[...]
    cnt = cnt_ref[...]
    col = lax.broadcasted_iota(jnp.int32, (NW, 16), 1)
    tcls = jnp.sum(jnp.where(col == 0, part, 0.0))
    tacc = jnp.sum(jnp.where(col == 1, part, 0.0))
    treg = jnp.sum(jnp.where(col == 2, part, 0.0))
    tot_p = jnp.sum(jnp.where(col == 0, cnt, 0.0))
    tot_n = jnp.sum(jnp.where(col == 1, cnt, 0.0))
    n_pos = jnp.minimum(tot_p, float(NPOS_CAP))
    n_sel = n_pos + jnp.minimum(tot_n, float(NNEG_CAP))
    cls_out[0, 0] = -tcls / n_sel
    acc_out[0, 0] = tacc / n_sel
    rl = treg / jnp.maximum(n_pos, 1.0) / 4.0
    reg_out[0, 0] = jnp.where(n_pos > 0.0, rl, 0.0)


def _final_stage(partials, cnts16):
    return pl.pallas_call(
        _final_kernel,
        in_specs=[
            pl.BlockSpec((NW, 16), lambda: (0, 0)),
            pl.BlockSpec((NW, 16), lambda: (0, 0)),
        ],
        out_specs=[
            pl.BlockSpec((1, 1), lambda: (0, 0), memory_space=pltpu.SMEM),
            pl.BlockSpec((1, 1), lambda: (0, 0), memory_space=pltpu.SMEM),
            pl.BlockSpec((1, 1), lambda: (0, 0), memory_space=pltpu.SMEM),
        ],
        out_shape=[jax.ShapeDtypeStruct((1, 1), jnp.float32)] * 3,
    )(partials, cnts16)


@jax.jit
def kernel(nms_reg, nms_cls, rcnn_reg, rcnn_cls, bboxes, classes):
    del nms_cls
    pad = ((0, 0), (0, 0), (0, RP - R))
    nms_t = jnp.pad(jnp.transpose(nms_reg, (0, 2, 1)), pad).reshape(B, 4, SUB, 128)

    mpos, mneg, bcls, bbox, cnts = _match_stage(nms_t, bboxes, classes)

    mpos = mpos.reshape(NW, CHUNK)
    mneg = mneg.reshape(NW, CHUNK)
    bcls = bcls.reshape(NW, CHUNK)
    bbox = jnp.transpose(bbox.reshape(B, 4, RP), (0, 2, 1)).reshape(NW, CHUNK, 4)
    rpad = ((0, 0), (0, RP - R), (0, 0))
    nms_p = jnp.pad(nms_reg, rpad).reshape(NW, CHUNK, 4)
    reg_p = jnp.pad(rcnn_reg, rpad).reshape(NW, CHUNK, 4)
    cls_p = rcnn_cls.reshape(FLAT_ROWS, 16)
    cnts16 = jnp.pad(cnts.reshape(NW, 2), ((0, 0), (0, 14)))

    partials = _sc_stage(mpos, mneg, bcls, bbox, nms_p, reg_p, cls_p, cnts16)
    cls_l, reg_l, acc_l = _final_stage(partials, cnts16)
    return (cls_l.reshape(1), reg_l.reshape(1), acc_l.reshape(1))


# speedup vs baseline: 1.2147x; 1.0457x over previous
import jax
import jax.numpy as jnp
from jax import lax
from jax.experimental import pallas as pl
from jax.experimental.pallas import tpu as pltpu
from jax.experimental.pallas import tpu_sc as plsc

TOP, LEFT, BOTTOM, RIGHT = 0, 1, 2, 3
REDUCTION = 16.0
B, T, R, C = 8, 100, 5000, 81
RP = 5120
SUB = 40
NW = 32
CHUNK = 1280
NBLK = CHUNK // 16
FLAT_ROWS = B * R * C // 16
NPOS_CAP = 128
NNEG_CAP = 384
LN2 = 0.6931471805599453
SQRT2 = 1.4142135623730951


def _match_kernel(nms_ref, bb_ref, cl_ref, mpos_out, mneg_out, bcls_out, bbox_out,
                  cnt_out):
    a_t = nms_ref[0, TOP]
    a_l = nms_ref[0, LEFT]
    a_b = nms_ref[0, BOTTOM]
    a_r = nms_ref[0, RIGHT]
    area_a = jnp.maximum(a_b - a_t, 0.0) * jnp.maximum(a_r - a_l, 0.0)

    def iou_step(t, carry):
        best_iou, best_cls, bb_t, bb_l, bb_b, bb_r = carry
        g_t = bb_ref[0, t, TOP]
        g_l = bb_ref[0, t, LEFT]
        g_b = bb_ref[0, t, BOTTOM]
        g_r = bb_ref[0, t, RIGHT]
        area_b = jnp.maximum(g_b - g_t, 0.0) * jnp.maximum(g_r - g_l, 0.0)
        it = jnp.maximum(a_t, g_t)
        il = jnp.maximum(a_l, g_l)
        ib = jnp.minimum(a_b, g_b)
        ir = jnp.minimum(a_r, g_r)
        inter = jnp.maximum(ib - it, 0.0) * jnp.maximum(ir - il, 0.0)
        union = area_a + area_b - inter
        iou = inter / jnp.maximum(union, 1e-8)
        upd = iou > best_iou
        return (jnp.where(upd, iou, best_iou),
                jnp.where(upd, cl_ref[0, 0, t], best_cls),
                jnp.where(upd, g_t, bb_t),
                jnp.where(upd, g_l, bb_l),
                jnp.where(upd, g_b, bb_b),
                jnp.where(upd, g_r, bb_r))

    carry = (jnp.full((SUB, 128), -1.0, jnp.float32),
             jnp.zeros((SUB, 128), jnp.int32),
             jnp.zeros((SUB, 128), jnp.float32),
             jnp.zeros((SUB, 128), jnp.float32),
             jnp.zeros((SUB, 128), jnp.float32),
             jnp.zeros((SUB, 128), jnp.float32))
    for t in range(T):
        carry = iou_step(t, carry)
    best_iou, best_cls, bb_t, bb_l, bb_b, bb_r = carry

    row = lax.broadcasted_iota(jnp.int32, (SUB, 128), 0)
    col = lax.broadcasted_iota(jnp.int32, (SUB, 128), 1)
    valid = (row * 128 + col) < R
    is_pos = best_iou > 0.5
    mp = (is_pos & valid).astype(jnp.int32)
    mn = ((~is_pos) & valid).astype(jnp.int32)
    mpos_out[0] = mp
    mneg_out[0] = mn
    bcls_out[0] = best_cls
    bbox_out[0, TOP] = bb_t
    bbox_out[0, LEFT] = bb_l
    bbox_out[0, BOTTOM] = bb_b
    bbox_out[0, RIGHT] = bb_r
    q = SUB // 4
    for i in range(4):
        cnt_out[0, 0, 2 * i] = jnp.sum(mp[i * q:(i + 1) * q, :].astype(jnp.float32))
        cnt_out[0, 0, 2 * i + 1] = jnp.sum(mn[i * q:(i + 1) * q, :].astype(jnp.float32))


def _match_stage(nms_t, bboxes, classes):
    return pl.pallas_call(
        _match_kernel,
        grid=(B,),
        in_specs=[
            pl.BlockSpec((1, 4, SUB, 128), lambda b: (b, 0, 0, 0)),
            pl.BlockSpec((1, T, 4), lambda b: (b, 0, 0), memory_space=pltpu.SMEM),
            pl.BlockSpec((1, 1, T), lambda b: (b, 0, 0), memory_space=pltpu.SMEM),
        ],
        out_specs=[
            pl.BlockSpec((1, SUB, 128), lambda b: (b, 0, 0)),
            pl.BlockSpec((1, SUB, 128), lambda b: (b, 0, 0)),
            pl.BlockSpec((1, SUB, 128), lambda b: (b, 0, 0)),
            pl.BlockSpec((1, 4, SUB, 128), lambda b: (b, 0, 0, 0)),
            pl.BlockSpec((1, 1, 8), lambda b: (b, 0, 0), memory_space=pltpu.SMEM),
        ],
        out_shape=[
            jax.ShapeDtypeStruct((B, SUB, 128), jnp.int32),
            jax.ShapeDtypeStruct((B, SUB, 128), jnp.int32),
            jax.ShapeDtypeStruct((B, SUB, 128), jnp.int32),
            jax.ShapeDtypeStruct((B, 4, SUB, 128), jnp.float32),
            jax.ShapeDtypeStruct((B, 1, 8), jnp.float32),
        ],
    )(nms_t, bboxes, classes.reshape(B, 1, T))


def _vlog(x):
    bits = plsc.bitcast(x, jnp.int32)
    e = ((bits >> 23) & 0xFF) - 127
    m = plsc.bitcast((bits & 0x7FFFFF) | 0x3F800000, jnp.float32)
    big = m > SQRT2
    m = jnp.where(big, m * 0.5, m)
    e = jnp.where(big, e + 1, e)
    s = (m - 1.0) / (m + 1.0)
    s2 = s * s
    p = 2.0 * s * (1.0 + s2 * (1.0 / 3.0 + s2 * (0.2 + s2 * (1.0 / 7.0 + s2 / 9.0))))
    return e.astype(jnp.float32) * LN2 + p


def _floor16(v):
    t = v.astype(jnp.int32).astype(jnp.float32)
    return t - (v < t).astype(jnp.float32)


def _ceil16(v):
    t = v.astype(jnp.int32).astype(jnp.float32)
    return t + (v > t).astype(jnp.float32)


def _splat_i(val):
    return jnp.full((16,), val, jnp.int32)


def _sc_kernel(mpos_hbm, mneg_hbm, bcls_hbm, bbox_hbm, nms_hbm, reg_hbm, cls_hbm,
               cnts_hbm,
               out_hbm,
               mpos_v, mneg_v, bcls_v, bbox_v, nms_v, reg_v,
               pos_list, neg_list, allcnt_v, rows_v, out_v, sem):
    wid = lax.axis_index("s") * 2 + lax.axis_index("c")
    iota = lax.iota(jnp.int32, 16)

    pltpu.sync_copy(mpos_hbm.at[pl.ds(wid * 10, 10)], mpos_v)
    pltpu.sync_copy(mneg_hbm.at[pl.ds(wid * 10, 10)], mneg_v)
    pltpu.sync_copy(bcls_hbm.at[pl.ds(wid * 10, 10)], bcls_v)
    pltpu.sync_copy(bbox_hbm.at[pl.ds(wid * 40, 40)], bbox_v)
    pltpu.sync_copy(nms_hbm.at[pl.ds(wid * 40, 40)], nms_v)
    pltpu.sync_copy(reg_hbm.at[pl.ds(wid * 40, 40)], reg_v)
    pltpu.sync_copy(cnts_hbm, allcnt_v)

    def compact(i, carry):
        cp, cn = carry
        mp = mpos_v[i >> 3, pl.ds((i & 7) * 16, 16)]
        mn = mneg_v[i >> 3, pl.ds((i & 7) * 16, 16)]
        jvec = iota + i * 16
        cump = plsc.cumsum(mp)
        cumn = plsc.cumsum(mn)
        plsc.store_scatter(pos_list, [cp + cump - mp], jvec, mask=mp != 0)
        plsc.store_scatter(neg_list, [cn + cumn - mn], jvec, mask=mn != 0)
        return (cp + jnp.sum(mp), cn + jnp.sum(mn))

    cnt_p, cnt_n = lax.fori_loop(0, NBLK, compact, (jnp.int32(0), jnp.int32(0)))

    pos_off = jnp.int32(0)
    neg_off = jnp.int32(0)
    for w2 in range(NW):
        cnt_row = allcnt_v[w2 >> 3, pl.ds((w2 & 7) * 16, 16)]
        cp = cnt_row[0].astype(jnp.int32)
        cn = cnt_row[1].astype(jnp.int32)
        before = jnp.int32(w2) < wid
        pos_off = pos_off + jnp.where(before, cp, 0)
        neg_off = neg_off + jnp.where(before, cn, 0)

    take_p = jnp.clip(NPOS_CAP - pos_off, 0, cnt_p)
    take_n = jnp.clip(NNEG_CAP - neg_off, 0, cnt_n)

    base = (wid >> 2) * R + (wid & 3) * CHUNK

    def row_block(lst_ref, take, is_pos_blk):
        def body(i, carry):
            cls_s, acc_s, reg_s = carry
            g = i * 16 + iota
            active = g < take
            j16 = lst_ref[pl.ds(i * 16, 16)]
            sj = jnp.where(active, j16, 0)
            gidx = sj + base
            e = gidx * C
            r0 = e >> 4
            o = e & 15
            copies = [pltpu.async_copy(cls_hbm.at[r0 + k], rows_v.at[k], sem)
                      for k in range(6)]
            for cp_ in copies:
                cp_.wait()
            cls16 = (plsc.load_gather(bcls_v, [sj >> 7, sj & 127])
                     if is_pos_blk else _splat_i(0))

            def col(c, cc):
                mx, se, am, xc = cc
                t = o + c
                cv = plsc.load_gather(rows_v, [t >> 4, iota, t & 15])
                gt = cv > mx
                mx2 = jnp.where(gt, cv, mx)
                se2 = se * jnp.exp(mx - mx2) + jnp.exp(cv - mx2)
                am2 = jnp.where(gt, c, am)
                xc2 = jnp.where(cls16 == c, cv, xc)
                return (mx2, se2, am2, xc2)

            mx0 = jnp.full((16,), -3.0e38, jnp.float32)
            z = jnp.zeros((16,), jnp.float32)
            mx, se, am, xc = lax.fori_loop(0, C, col, (mx0, z, _splat_i(0), z))
            lp = xc - mx - _vlog(se)
            af = jnp.where(active, 1.0, 0.0)
            cls_s = cls_s + jnp.sum(af * lp)
            hit = (am == cls16).astype(jnp.float32)
            acc_s = acc_s + jnp.sum(af * hit)

            if is_pos_blk:
                for c4, is_ceil in ((TOP, 0), (LEFT, 0), (BOTTOM, 1), (RIGHT, 1)):
                    f4 = sj * 4 + c4
                    fr, fc = f4 >> 7, f4 & 127
                    nv = plsc.load_gather(nms_v, [fr, fc]) * REDUCTION
                    rounded = (_ceil16(nv) if is_ceil else _floor16(nv)) / REDUCTION
                    rg = plsc.load_gather(reg_v, [fr, fc])
                    bb = plsc.load_gather(bbox_v, [fr, fc])
                    d = jnp.abs(rg - (bb - rounded))
                    term = jnp.where(d < 1.0, 0.5 * d * d, d - 0.5)
                    reg_s = reg_s + jnp.sum(af * term)
            return (cls_s, acc_s, reg_s)

        nblocks = (take + 15) >> 4
        z3 = (jnp.float32(0.0), jnp.float32(0.0), jnp.float32(0.0))
        return lax.fori_loop(0, nblocks, body, z3)

    pc, pa, pr = row_block(pos_list, take_p, True)
    nc, na, _ = row_block(neg_list, take_n, False)

    sums = (jnp.where(iota == 0, pc + nc,
            jnp.where(iota == 1, pa + na,
            jnp.where(iota == 2, pr, 0.0))))
    out_v[...] = sums
    pltpu.sync_copy(out_v, out_hbm.at[wid >> 3, pl.ds((wid & 7) * 16, 16)])


def _sc_stage(mpos, mneg, bcls, bbox, nms_p, reg_p, cls_p, cnts16):
    mesh = plsc.VectorSubcoreMesh(core_axis_name="c", subcore_axis_name="s")
    import functools
    k = functools.partial(
        pl.kernel,
        out_type=jax.ShapeDtypeStruct((NW // 8, 128), jnp.float32),
        mesh=mesh,
        compiler_params=pltpu.CompilerParams(
            needs_layout_passes=False, use_tc_tiling_on_sc=False),
        scratch_types=[
            pltpu.VMEM((10, 128), jnp.int32),
            pltpu.VMEM((10, 128), jnp.int32),
            pltpu.VMEM((10, 128), jnp.int32),
            pltpu.VMEM((40, 128), jnp.float32),
            pltpu.VMEM((40, 128), jnp.float32),
            pltpu.VMEM((40, 128), jnp.float32),
            pltpu.VMEM((CHUNK,), jnp.int32),
            pltpu.VMEM((CHUNK,), jnp.int32),
            pltpu.VMEM((NW // 8, 128), jnp.float32),
            pltpu.VMEM((6, 16, 16), jnp.float32),
            pltpu.VMEM((16,), jnp.float32),
            pltpu.SemaphoreType.DMA,
        ],
    )(_sc_kernel)
    return k(mpos, mneg, bcls, bbox, nms_p, reg_p, cls_p, cnts16)


def _final_kernel(part_ref, cnt_ref, cls_out, reg_out, acc_out):
    part = part_ref[...]
    cnt = cnt_ref[...]
    col = lax.broadcasted_iota(jnp.int32, (NW // 8, 128), 1) & 15
    tcls = jnp.sum(jnp.where(col == 0, part, 0.0))
    tacc = jnp.sum(jnp.where(col == 1, part, 0.0))
    treg = jnp.sum(jnp.where(col == 2, part, 0.0))
    tot_p = jnp.sum(jnp.where(col == 0, cnt, 0.0))
    tot_n = jnp.sum(jnp.where(col == 1, cnt, 0.0))
    n_pos = jnp.minimum(tot_p, float(NPOS_CAP))
    n_sel = n_pos + jnp.minimum(tot_n, float(NNEG_CAP))
    cls_out[0, 0] = -tcls / n_sel
    acc_out[0, 0] = tacc / n_sel
    rl = treg / jnp.maximum(n_pos, 1.0) / 4.0
    reg_out[0, 0] = jnp.where(n_pos > 0.0, rl, 0.0)


def _final_stage(partials, cnts16):
    return pl.pallas_call(
        _final_kernel,
        in_specs=[
            pl.BlockSpec((NW // 8, 128), lambda: (0, 0)),
            pl.BlockSpec((NW // 8, 128), lambda: (0, 0)),
        ],
        out_specs=[
            pl.BlockSpec((1, 1), lambda: (0, 0), memory_space=pltpu.SMEM),
            pl.BlockSpec((1, 1), lambda: (0, 0), memory_space=pltpu.SMEM),
            pl.BlockSpec((1, 1), lambda: (0, 0), memory_space=pltpu.SMEM),
        ],
        out_shape=[jax.ShapeDtypeStruct((1, 1), jnp.float32)] * 3,
    )(partials, cnts16)


@jax.jit
def kernel(nms_reg, nms_cls, rcnn_reg, rcnn_cls, bboxes, classes):
    del nms_cls
    pad = ((0, 0), (0, 0), (0, RP - R))
    nms_t = jnp.pad(jnp.transpose(nms_reg, (0, 2, 1)), pad).reshape(B, 4, SUB, 128)

    mpos, mneg, bcls, bbox, cnts = _match_stage(nms_t, bboxes, classes)

    mpos = mpos.reshape(NW * 10, 128)
    mneg = mneg.reshape(NW * 10, 128)
    bcls = bcls.reshape(NW * 10, 128)
    bbox = jnp.transpose(bbox.reshape(B, 4, RP), (0, 2, 1)).reshape(NW * 40, 128)
    rpad = ((0, 0), (0, RP - R), (0, 0))
    nms_p = jnp.pad(nms_reg, rpad).reshape(NW * 40, 128)
    reg_p = jnp.pad(rcnn_reg, rpad).reshape(NW * 40, 128)
    cls_p = rcnn_cls.reshape(FLAT_ROWS, 16)
    cnts16 = jnp.pad(cnts.reshape(NW, 2), ((0, 0), (0, 14))).reshape(NW // 8, 128)

    partials = _sc_stage(mpos, mneg, bcls, bbox, nms_p, reg_p, cls_p, cnts16)
    cls_l, reg_l, acc_l = _final_stage(partials, cnts16)
    return (cls_l.reshape(1), reg_l.reshape(1), acc_l.reshape(1))


# speedup vs baseline: 1.6788x; 1.3820x over previous
import jax
import jax.numpy as jnp
from jax import lax
from jax.experimental import pallas as pl
from jax.experimental.pallas import tpu as pltpu
from jax.experimental.pallas import tpu_sc as plsc

TOP, LEFT, BOTTOM, RIGHT = 0, 1, 2, 3
REDUCTION = 16.0
B, T, R, C = 8, 100, 5000, 81
RP = 5120
SUB = 40
NW = 32
CHUNK = 1280
NBLK = CHUNK // 16
FLAT_ROWS = B * R * C // 16
NPOS_CAP = 128
NNEG_CAP = 384
LN2 = 0.6931471805599453
SQRT2 = 1.4142135623730951


def _match_kernel(nms_ref, bb_ref, cl_ref, enc_out, cnt_out):
    a_t = nms_ref[0, TOP]
    a_l = nms_ref[0, LEFT]
    a_b = nms_ref[0, BOTTOM]
    a_r = nms_ref[0, RIGHT]
    area_a = jnp.maximum(a_b - a_t, 0.0) * jnp.maximum(a_r - a_l, 0.0)

    def iou_step(t, carry):
        best_iou, best_cls, best_t = carry
        g_t = bb_ref[0, t, TOP]
        g_l = bb_ref[0, t, LEFT]
        g_b = bb_ref[0, t, BOTTOM]
        g_r = bb_ref[0, t, RIGHT]
        area_b = jnp.maximum(g_b - g_t, 0.0) * jnp.maximum(g_r - g_l, 0.0)
        it = jnp.maximum(a_t, g_t)
        il = jnp.maximum(a_l, g_l)
        ib = jnp.minimum(a_b, g_b)
        ir = jnp.minimum(a_r, g_r)
        inter = jnp.maximum(ib - it, 0.0) * jnp.maximum(ir - il, 0.0)
        union = area_a + area_b - inter
        iou = inter / jnp.maximum(union, 1e-8)
        upd = iou > best_iou
        return (jnp.where(upd, iou, best_iou),
                jnp.where(upd, cl_ref[0, 0, t], best_cls),
                jnp.where(upd, t, best_t))

    carry = (jnp.full((SUB, 128), -1.0, jnp.float32),
             jnp.zeros((SUB, 128), jnp.int32),
             jnp.zeros((SUB, 128), jnp.int32))
    for t in range(T):
        carry = iou_step(t, carry)
    best_iou, best_cls, best_t = carry

    row = lax.broadcasted_iota(jnp.int32, (SUB, 128), 0)
    col = lax.broadcasted_iota(jnp.int32, (SUB, 128), 1)
    valid = (row * 128 + col) < R
    is_pos = best_iou > 0.5
    mp = (is_pos & valid).astype(jnp.int32)
    mn = ((~is_pos) & valid).astype(jnp.int32)
    enc_out[0] = best_cls | (mp << 8) | (mn << 9) | (best_t << 16)
    q = SUB // 4
    for i in range(4):
        cnt_out[0, 0, 2 * i] = jnp.sum(mp[i * q:(i + 1) * q, :].astype(jnp.float32))
        cnt_out[0, 0, 2 * i + 1] = jnp.sum(mn[i * q:(i + 1) * q, :].astype(jnp.float32))


def _match_stage(nms_t, bboxes, classes):
    return pl.pallas_call(
        _match_kernel,
        grid=(B,),
        in_specs=[
            pl.BlockSpec((1, 4, SUB, 128), lambda b: (b, 0, 0, 0)),
            pl.BlockSpec((1, T, 4), lambda b: (b, 0, 0), memory_space=pltpu.SMEM),
            pl.BlockSpec((1, 1, T), lambda b: (b, 0, 0), memory_space=pltpu.SMEM),
        ],
        out_specs=[
            pl.BlockSpec((1, SUB, 128), lambda b: (b, 0, 0)),
            pl.BlockSpec((1, 1, 8), lambda b: (b, 0, 0), memory_space=pltpu.SMEM),
        ],
        out_shape=[
            jax.ShapeDtypeStruct((B, SUB, 128), jnp.int32),
            jax.ShapeDtypeStruct((B, 1, 8), jnp.float32),
        ],
    )(nms_t, bboxes, classes.reshape(B, 1, T))


def _vlog(x):
    bits = plsc.bitcast(x, jnp.int32)
    e = ((bits >> 23) & 0xFF) - 127
    m = plsc.bitcast((bits & 0x7FFFFF) | 0x3F800000, jnp.float32)
    big = m > SQRT2
    m = jnp.where(big, m * 0.5, m)
    e = jnp.where(big, e + 1, e)
    s = (m - 1.0) / (m + 1.0)
    s2 = s * s
    p = 2.0 * s * (1.0 + s2 * (1.0 / 3.0 + s2 * (0.2 + s2 * (1.0 / 7.0 + s2 / 9.0))))
    return e.astype(jnp.float32) * LN2 + p


def _floor16(v):
    t = v.astype(jnp.int32).astype(jnp.float32)
    return t - (v < t).astype(jnp.float32)


def _ceil16(v):
    t = v.astype(jnp.int32).astype(jnp.float32)
    return t + (v > t).astype(jnp.float32)


def _splat_i(val):
    return jnp.full((16,), val, jnp.int32)


def _sc_kernel(enc_hbm, nms_hbm, reg_hbm, bbf_hbm, cls_hbm, cnts_hbm,
               out_hbm,
               enc_v, pos_list, neg_list, allcnt_v, rows_v,
               nbuf, rbuf, bbuf, out_v, sem):
    wid = lax.axis_index("s") * 2 + lax.axis_index("c")
    iota = lax.iota(jnp.int32, 16)

    pltpu.sync_copy(enc_hbm.at[pl.ds(wid * 10, 10)], enc_v)
    pltpu.sync_copy(cnts_hbm, allcnt_v)

    def compact(i, carry):
        cp, cn = carry
        e16 = enc_v[i >> 3, pl.ds((i & 7) * 16, 16)]
        mp = (e16 >> 8) & 1
        mn = (e16 >> 9) & 1
        jvec = iota + i * 16
        cump = plsc.cumsum(mp)
        cumn = plsc.cumsum(mn)
        plsc.store_scatter(pos_list, [cp + cump - mp], jvec, mask=mp != 0)
        plsc.store_scatter(neg_list, [cn + cumn - mn], jvec, mask=mn != 0)
        return (cp + jnp.sum(mp), cn + jnp.sum(mn))

    cnt_p, cnt_n = lax.fori_loop(0, NBLK, compact, (jnp.int32(0), jnp.int32(0)))

    pos_off = jnp.int32(0)
    neg_off = jnp.int32(0)
    for w2 in range(NW):
        cnt_row = allcnt_v[w2 >> 3, pl.ds((w2 & 7) * 16, 16)]
        cp = cnt_row[0].astype(jnp.int32)
        cn = cnt_row[1].astype(jnp.int32)
        before = jnp.int32(w2) < wid
        pos_off = pos_off + jnp.where(before, cp, 0)
        neg_off = neg_off + jnp.where(before, cn, 0)

    take_p = jnp.clip(NPOS_CAP - pos_off, 0, cnt_p)
    take_n = jnp.clip(NNEG_CAP - neg_off, 0, cnt_n)

    base = (wid >> 2) * R + (wid & 3) * CHUNK

    def row_block(lst_ref, take, is_pos_blk):
        def body(i, carry):
            cls_s, acc_s, reg_s = carry
            g = i * 16 + iota
            active = g < take
            j16 = lst_ref[pl.ds(i * 16, 16)]
            sj = jnp.where(active, j16, 0)
            gidx = sj + base
            e = gidx * C
            r0 = e >> 4
            o = e & 15
            copies = [pltpu.async_copy(cls_hbm.at[r0 + k], rows_v.at[k], sem)
                      for k in range(6)]
            if is_pos_blk:
                encj = plsc.load_gather(enc_v, [sj >> 7, sj & 127])
                cls16 = encj & 0x7F
                t16 = (encj >> 16) & 0x7F
                gn = gidx * 4
                rn0, on = gn >> 4, gn & 15
                gb = ((wid >> 2) * T + t16) * 4
                rb0, ob = gb >> 4, gb & 15
                copies.append(pltpu.async_copy(nms_hbm.at[rn0], nbuf, sem))
                copies.append(pltpu.async_copy(reg_hbm.at[rn0], rbuf, sem))
                copies.append(pltpu.async_copy(bbf_hbm.at[rb0], bbuf, sem))
            else:
                cls16 = _splat_i(0)
            for cp_ in copies:
                cp_.wait()

            def col(c, cc):
                mx, se, am, xc = cc
                t = o + c
                cv = plsc.load_gather(rows_v, [t >> 4, iota, t & 15])
                gt = cv > mx
                mx2 = jnp.where(gt, cv, mx)
                se2 = se * jnp.exp(mx - mx2) + jnp.exp(cv - mx2)
                am2 = jnp.where(gt, c, am)
                xc2 = jnp.where(cls16 == c, cv, xc)
                return (mx2, se2, am2, xc2)

            mx0 = jnp.full((16,), -3.0e38, jnp.float32)
            z = jnp.zeros((16,), jnp.float32)
            mx, se, am, xc = lax.fori_loop(0, C, col, (mx0, z, _splat_i(0), z))
            lp = xc - mx - _vlog(se)
            af = jnp.where(active, 1.0, 0.0)
            cls_s = cls_s + jnp.sum(af * lp)
            hit = (am == cls16).astype(jnp.float32)
            acc_s = acc_s + jnp.sum(af * hit)

            if is_pos_blk:
                for c4, is_ceil in ((TOP, 0), (LEFT, 0), (BOTTOM, 1), (RIGHT, 1)):
                    nv = plsc.load_gather(nbuf, [iota, on + c4]) * REDUCTION
                    rounded = (_ceil16(nv) if is_ceil else _floor16(nv)) / REDUCTION
                    rg = plsc.load_gather(rbuf, [iota, on + c4])
                    bb = plsc.load_gather(bbuf, [iota, ob + c4])
                    d = jnp.abs(rg - (bb - rounded))
                    term = jnp.where(d < 1.0, 0.5 * d * d, d - 0.5)
                    reg_s = reg_s + jnp.sum(af * term)
            return (cls_s, acc_s, reg_s)

        nblocks = (take + 15) >> 4
        z3 = (jnp.float32(0.0), jnp.float32(0.0), jnp.float32(0.0))
        return lax.fori_loop(0, nblocks, body, z3)

    pc, pa, pr = row_block(pos_list, take_p, True)
    nc, na, _ = row_block(neg_list, take_n, False)

    sums = (jnp.where(iota == 0, pc + nc,
            jnp.where(iota == 1, pa + na,
            jnp.where(iota == 2, pr, 0.0))))
    out_v[...] = sums
    pltpu.sync_copy(out_v, out_hbm.at[wid >> 3, pl.ds((wid & 7) * 16, 16)])


def _sc_stage(enc, nms_f, reg_f, bb_f, cls_p, cnts16):
    mesh = plsc.VectorSubcoreMesh(core_axis_name="c", subcore_axis_name="s")
    import functools
    k = functools.partial(
        pl.kernel,
        out_type=jax.ShapeDtypeStruct((NW // 8, 128), jnp.float32),
        mesh=mesh,
        compiler_params=pltpu.CompilerParams(
            needs_layout_passes=False, use_tc_tiling_on_sc=False),
        scratch_types=[
            pltpu.VMEM((10, 128), jnp.int32),
            pltpu.VMEM((CHUNK,), jnp.int32),
            pltpu.VMEM((CHUNK,), jnp.int32),
            pltpu.VMEM((NW // 8, 128), jnp.float32),
            pltpu.VMEM((6, 16, 16), jnp.float32),
            pltpu.VMEM((16, 16), jnp.float32),
            pltpu.VMEM((16, 16), jnp.float32),
            pltpu.VMEM((16, 16), jnp.float32),
            pltpu.VMEM((16,), jnp.float32),
            pltpu.SemaphoreType.DMA,
        ],
    )(_sc_kernel)
    return k(enc, nms_f, reg_f, bb_f, cls_p, cnts16)


def _final_kernel(part_ref, cnt_ref, cls_out, reg_out, acc_out):
    part = part_ref[...]
    cnt = cnt_ref[...]
    col = lax.broadcasted_iota(jnp.int32, (NW // 8, 128), 1) & 15
    tcls = jnp.sum(jnp.where(col == 0, part, 0.0))
    tacc = jnp.sum(jnp.where(col == 1, part, 0.0))
    treg = jnp.sum(jnp.where(col == 2, part, 0.0))
    tot_p = jnp.sum(jnp.where(col == 0, cnt, 0.0))
    tot_n = jnp.sum(jnp.where(col == 1, cnt, 0.0))
    n_pos = jnp.minimum(tot_p, float(NPOS_CAP))
    n_sel = n_pos + jnp.minimum(tot_n, float(NNEG_CAP))
    cls_out[0, 0] = -tcls / n_sel
    acc_out[0, 0] = tacc / n_sel
    rl = treg / jnp.maximum(n_pos, 1.0) / 4.0
    reg_out[0, 0] = jnp.where(n_pos > 0.0, rl, 0.0)


def _final_stage(partials, cnts16):
    return pl.pallas_call(
        _final_kernel,
        in_specs=[
            pl.BlockSpec((NW // 8, 128), lambda: (0, 0)),
            pl.BlockSpec((NW // 8, 128), lambda: (0, 0)),
        ],
        out_specs=[
            pl.BlockSpec((1, 1), lambda: (0, 0), memory_space=pltpu.SMEM),
            pl.BlockSpec((1, 1), lambda: (0, 0), memory_space=pltpu.SMEM),
            pl.BlockSpec((1, 1), lambda: (0, 0), memory_space=pltpu.SMEM),
        ],
        out_shape=[jax.ShapeDtypeStruct((1, 1), jnp.float32)] * 3,
    )(partials, cnts16)


@jax.jit
def kernel(nms_reg, nms_cls, rcnn_reg, rcnn_cls, bboxes, classes):
    del nms_cls
    pad = ((0, 0), (0, 0), (0, RP - R))
    nms_t = jnp.pad(jnp.transpose(nms_reg, (0, 2, 1)), pad).reshape(B, 4, SUB, 128)

    enc, cnts = _match_stage(nms_t, bboxes, classes)

    enc = enc.reshape(NW * 10, 128)
    nms_f = nms_reg.reshape(B * R * 4 // 16, 16)
    reg_f = rcnn_reg.reshape(B * R * 4 // 16, 16)
    bb_f = bboxes.reshape(B * T * 4 // 16, 16)
    cls_p = rcnn_cls.reshape(FLAT_ROWS, 16)
    cnts16 = jnp.pad(cnts.reshape(NW, 2), ((0, 0), (0, 14))).reshape(NW // 8, 128)

    partials = _sc_stage(enc, nms_f, reg_f, bb_f, cls_p, cnts16)
    cls_l, reg_l, acc_l = _final_stage(partials, cnts16)
    return (cls_l.reshape(1), reg_l.reshape(1), acc_l.reshape(1))
